# baseline (speedup 1.0000x reference)
import functools
import jax
import jax.numpy as jnp
from jax.experimental import pallas as pl
from jax.experimental.pallas import tpu as pltpu
from jax.experimental.pallas import tpu_sc as plsc


def kernel(x, emb_table):
    B, T, D = x.shape
    BR = 16
    NT = T // BR
    L = 16
    blk = BR * D

    xf = x.reshape(B * T * D)
    ef = emb_table.reshape(T * D)

    mesh = plsc.VectorSubcoreMesh(core_axis_name="c", subcore_axis_name="s")

    @functools.partial(
        pl.kernel,
        out_type=jax.ShapeDtypeStruct((B * T * D,), x.dtype),
        mesh=mesh,
    )
    def sc_add(x_hbm, e_hbm, o_hbm):
        def body(x_v, e_v, o_v):
            @pl.loop(0, blk, step=L)
            def _(c):
                o_v.at[pl.ds(c, L)][...] = (
                    x_v.at[pl.ds(c, L)][...] + e_v.at[pl.ds(c, L)][...]
                )

        pltpu.emit_pipeline(
            body,
            grid=(NT, B),
            in_specs=[
                pl.BlockSpec((blk,), index_map=lambda t, b: (b * NT + t,)),
                pl.BlockSpec((blk,), index_map=lambda t, b: (t,)),
            ],
            out_specs=[pl.BlockSpec((blk,), index_map=lambda t, b: (b * NT + t,))],
            core_axis_name=("c", "s"),
            dimension_semantics=(pltpu.PARALLEL, pltpu.PARALLEL),
        )(x_hbm, e_hbm, o_hbm)

    return sc_add(xf, ef).reshape(B, T, D)


# speedup vs baseline: 7.4695x; 7.4695x over previous
import jax
import jax.numpy as jnp
from jax.experimental import pallas as pl


def _add_body(x_ref, e_ref, o_ref):
    o_ref[...] = x_ref[...] + e_ref[...][None]


def kernel(x, emb_table):
    B, T, D = x.shape
    BT = 2048
    BB = 1
    return pl.pallas_call(
        _add_body,
        grid=(T // BT, B // BB),
        in_specs=[
            pl.BlockSpec((BB, BT, D), lambda i, j: (j, i, 0)),
            pl.BlockSpec((BT, D), lambda i, j: (i, 0)),
        ],
        out_specs=pl.BlockSpec((BB, BT, D), lambda i, j: (j, i, 0)),
        out_shape=jax.ShapeDtypeStruct(x.shape, x.dtype),
    )(x, emb_table)


# speedup vs baseline: 7.4757x; 1.0008x over previous
import jax
import jax.numpy as jnp
from jax.experimental import pallas as pl
from jax.experimental.pallas import tpu as pltpu


def _add_body(x_ref, e_ref, o_ref):
    o_ref[...] = x_ref[...] + e_ref[...][None]


def kernel(x, emb_table):
    B, T, D = x.shape
    BT = 2048
    BB = 1
    return pl.pallas_call(
        _add_body,
        grid=(T // BT, B // BB),
        in_specs=[
            pl.BlockSpec((BB, BT, D), lambda i, j: (j, i, 0)),
            pl.BlockSpec((BT, D), lambda i, j: (i, 0)),
        ],
        out_specs=pl.BlockSpec((BB, BT, D), lambda i, j: (j, i, 0)),
        out_shape=jax.ShapeDtypeStruct(x.shape, x.dtype),
        compiler_params=pltpu.CompilerParams(
            dimension_semantics=("parallel", "parallel"),
        ),
    )(x, emb_table)
